# baseline (speedup 1.0000x reference)
import jax
import jax.numpy as jnp
from jax.experimental import pallas as pl

N = 10000
G = 64
P = 2


def kernel(x, edge_index, batch, W_emb, b_emb, W0, bW0, g0, bt0, a0, W1, bW1, g1, bt1, a1, W2, bW2, g2, bt2, a2):
    layers = [(W0, bW0, g0, bt0, a0), (W1, bW1, g1, bt1, a1), (W2, bW2, g2, bt2, a2)]
    src = jnp.concatenate([edge_index[0], jnp.arange(N, dtype=edge_index.dtype)])
    dst = jnp.concatenate([edge_index[1], jnp.arange(N, dtype=edge_index.dtype)])
    deg = jax.ops.segment_sum(jnp.ones(src.shape[0], jnp.float32), dst, num_segments=N)
    dinv = 1.0 / jnp.sqrt(jnp.clip(deg, 1.0))
    norm = (dinv[src] * dinv[dst])[:, None]
    cnt = jnp.maximum(jax.ops.segment_sum(jnp.ones((N,), jnp.float32), batch, num_segments=G), 1.0)[:, None]
    h = x @ W_emb + b_emb
    feats = []
    for (W, bW, g, bt, a) in layers:
        h = h @ W
        for _ in range(P):
            h = jax.ops.segment_sum(h[src] * norm, dst, num_segments=N)
        h = h + bW
        mean = jax.ops.segment_sum(h, batch, num_segments=G) / cnt
        sub = h - a * mean[batch]
        var = jax.ops.segment_sum(sub * sub, batch, num_segments=G) / cnt
        h = g * sub / jnp.sqrt(var[batch] + 1e-5) + bt
        h = jax.nn.leaky_relu(h, 0.01)
        feats.append(jax.ops.segment_sum(h, batch, num_segments=G) / cnt)
    return (jnp.mean(jnp.stack(feats, 0), 0), 0)


# speedup vs baseline: 8.9115x; 8.9115x over previous
import jax
import jax.numpy as jnp
from jax import lax
from jax.experimental import pallas as pl
from jax.experimental.pallas import tpu as pltpu
from jax.experimental.pallas import tpu_sc as plsc

_N = 10000
_E = 320000
_H = 128
_G = 64
_NC = 2
_NS = 16
_NW = _NC * _NS
_CH = 128
_STEPS = 79
_EPAD = _NW * _STEPS * _CH
_NP = 10240
_RPS = _NP // _NS

_MESH = plsc.VectorSubcoreMesh(core_axis_name="c", subcore_axis_name="s",
                               num_cores=_NC, num_subcores=_NS)
_HIGH = lax.Precision.HIGHEST


def _deg_body(dst_hbm, ones_hbm, zeros_hbm, out_hbm, dstv, onesv, dacc):
    c = lax.axis_index("c")
    s = lax.axis_index("s")
    w = c * _NS + s
    for t in range(_RPS // _CH):
        pltpu.sync_copy(zeros_hbm, dacc.at[pl.ds(s * _RPS + t * _CH, _CH)])
    pltpu.sync_copy(dst_hbm.at[w], dstv)
    pltpu.sync_copy(ones_hbm, onesv)
    plsc.subcore_barrier()

    def step(j, carry):
        pltpu.sync_copy(onesv, dacc.at[dstv.at[j]], add=True)
        return carry

    lax.fori_loop(0, _STEPS, step, 0)
    plsc.subcore_barrier()
    pltpu.sync_copy(dacc.at[pl.ds(s * _RPS, _RPS)],
                    out_hbm.at[c, pl.ds(s * _RPS, _RPS)])


_deg = pl.kernel(
    _deg_body,
    out_type=jax.ShapeDtypeStruct((_NC, _NP, _H), jnp.float32),
    mesh=_MESH,
    scratch_types=[
        pltpu.VMEM((_STEPS, _CH), jnp.int32),
        pltpu.VMEM((_CH, _H), jnp.float32),
        pltpu.VMEM_SHARED((_NP, _H), jnp.float32),
    ],
)


def _prop_body(u_hbm, src_hbm, dst_hbm, zeros_hbm, out_hbm,
               srcv, dstv, rows, acc, sem):
    c = lax.axis_index("c")
    s = lax.axis_index("s")
    w = c * _NS + s
    for t in range(_RPS // _CH):
        pltpu.sync_copy(zeros_hbm, acc.at[pl.ds(s * _RPS + t * _CH, _CH)])
    pltpu.sync_copy(src_hbm.at[w], srcv)
    pltpu.sync_copy(dst_hbm.at[w], dstv)
    plsc.subcore_barrier()

    def step(j, carry):
        pltpu.async_copy(u_hbm.at[srcv.at[j]], rows, sem).wait()
        pltpu.sync_copy(rows, acc.at[dstv.at[j]], add=True)
        return carry

    lax.fori_loop(0, _STEPS, step, 0)
    plsc.subcore_barrier()
    pltpu.sync_copy(acc.at[pl.ds(s * _RPS, _RPS)],
                    out_hbm.at[c, pl.ds(s * _RPS, _RPS)])


_prop = pl.kernel(
    _prop_body,
    out_type=jax.ShapeDtypeStruct((_NC, _NP, _H), jnp.float32),
    mesh=_MESH,
    scratch_types=[
        pltpu.VMEM((_STEPS, _CH), jnp.int32),
        pltpu.VMEM((_STEPS, _CH), jnp.int32),
        pltpu.VMEM((_CH, _H), jnp.float32),
        pltpu.VMEM_SHARED((_NP, _H), jnp.float32),
        pltpu.SemaphoreType.DMA,
    ],
)


def _dot(a, b):
    return lax.dot_general(a, b, (((a.ndim - 1,), (0,)), ((), ())),
                           precision=_HIGH, preferred_element_type=jnp.float32)


def _tdot(a, b):
    return lax.dot_general(a, b, (((0,), (0,)), ((), ())),
                           precision=_HIGH, preferred_element_type=jnp.float32)


def _pre_body(x_ref, we_ref, be_ref, w0_ref, dp_ref, u_ref, dinv_ref):
    deg = dp_ref[0, 0:_N, 0:1] + dp_ref[1, 0:_N, 0:1] + 1.0
    dinv = lax.rsqrt(deg)
    dinv_ref[...] = dinv
    h = _dot(x_ref[...], we_ref[...]) + be_ref[...][None, :]
    u_ref[...] = dinv * _dot(h, w0_ref[...])


def _mid_body(sp_ref, u_ref, dinv_ref, out_ref):
    dinv = dinv_ref[...]
    out_ref[...] = dinv * dinv * (sp_ref[0, 0:_N] + sp_ref[1, 0:_N] + u_ref[...])


def _onehot(batch_ref):
    return (batch_ref[...][:, None] ==
            lax.broadcasted_iota(jnp.int32, (_N, _G), 1)).astype(jnp.float32)


def _stat_body(sp_ref, u_ref, dinv_ref, batch_ref, bw_ref, a_ref,
               h_ref, amean_ref, scale_ref):
    h = dinv_ref[...] * (sp_ref[0, 0:_N] + sp_ref[1, 0:_N] + u_ref[...]) \
        + bw_ref[...][None, :]
    h_ref[...] = h
    onehot = _onehot(batch_ref)
    cnt = jnp.maximum(jnp.sum(onehot, axis=0), 1.0)[:, None]
    amean = a_ref[...][None, :] * (_tdot(onehot, h) / cnt)
    amean_ref[...] = amean
    sub = h - _dot(onehot, amean)
    var = _tdot(onehot, sub * sub) / cnt
    scale_ref[...] = lax.rsqrt(var + 1e-5)


def _apply_core(h_ref, amean_ref, scale_ref, batch_ref, g_ref, bt_ref,
                fin_ref):
    onehot = _onehot(batch_ref)
    cnt = jnp.maximum(jnp.sum(onehot, axis=0), 1.0)[:, None]
    sub = h_ref[...] - _dot(onehot, amean_ref[...])
    hn = g_ref[...][None, :] * sub * _dot(onehot, scale_ref[...]) \
        + bt_ref[...][None, :]
    hn = jnp.where(hn >= 0, hn, 0.01 * hn)
    feat = _tdot(onehot, hn) / cnt
    return hn, fin_ref[...] + feat * (1.0 / 3.0)


def _apply_body(h_ref, amean_ref, scale_ref, batch_ref, g_ref, bt_ref,
                fin_ref, dinv_ref, wn_ref, fout_ref, unext_ref):
    hn, fout = _apply_core(h_ref, amean_ref, scale_ref, batch_ref, g_ref,
                           bt_ref, fin_ref)
    fout_ref[...] = fout
    unext_ref[...] = dinv_ref[...] * _dot(hn, wn_ref[...])


def _last_body(h_ref, amean_ref, scale_ref, batch_ref, g_ref, bt_ref,
               fin_ref, fout_ref):
    _, fout = _apply_core(h_ref, amean_ref, scale_ref, batch_ref, g_ref,
                          bt_ref, fin_ref)
    fout_ref[...] = fout


_f32 = jnp.float32
_pre = pl.pallas_call(
    _pre_body,
    out_shape=[jax.ShapeDtypeStruct((_N, _H), _f32),
               jax.ShapeDtypeStruct((_N, 1), _f32)])
_mid = pl.pallas_call(
    _mid_body,
    out_shape=jax.ShapeDtypeStruct((_N, _H), _f32))
_stat = pl.pallas_call(
    _stat_body,
    out_shape=[jax.ShapeDtypeStruct((_N, _H), _f32),
               jax.ShapeDtypeStruct((_G, _H), _f32),
               jax.ShapeDtypeStruct((_G, _H), _f32)])
_apply = pl.pallas_call(
    _apply_body,
    out_shape=[jax.ShapeDtypeStruct((_G, _H), _f32),
               jax.ShapeDtypeStruct((_N, _H), _f32)])
_last = pl.pallas_call(
    _last_body,
    out_shape=jax.ShapeDtypeStruct((_G, _H), _f32))


def _prep_edges(edge_index):
    pad = _EPAD - _E
    src = jnp.concatenate([edge_index[0], jnp.zeros((pad,), jnp.int32)])
    dst = jnp.concatenate([edge_index[1],
                           _N + (jnp.arange(pad, dtype=jnp.int32) % (_NP - _N))])
    src3 = src.reshape(_STEPS, _NW, _CH).transpose(1, 0, 2)
    dst3 = dst.reshape(_STEPS, _NW, _CH).transpose(1, 0, 2)
    return src3, dst3


def kernel(x, edge_index, batch, W_emb, b_emb, W0, bW0, g0, bt0, a0,
           W1, bW1, g1, bt1, a1, W2, bW2, g2, bt2, a2):
    src3, dst3 = _prep_edges(edge_index)
    ones_d = jnp.ones((_CH, _H), _f32)
    zeros_f = jnp.zeros((_CH, _H), _f32)

    dparts = _deg(dst3, ones_d, zeros_f)
    u, dinv = _pre(x, W_emb, b_emb, W0, dparts)

    feat = jnp.zeros((_G, _H), _f32)
    params = [(bW0, g0, bt0, a0, W1), (bW1, g1, bt1, a1, W2),
              (bW2, g2, bt2, a2, None)]
    for i, (bW, g, bt, a, Wn) in enumerate(params):
        sp = _prop(u, src3, dst3, zeros_f)
        u2 = _mid(sp, u, dinv)
        sp2 = _prop(u2, src3, dst3, zeros_f)
        h, amean, scale = _stat(sp2, u2, dinv, batch, bW, a)
        if Wn is not None:
            feat, u = _apply(h, amean, scale, batch, g, bt, feat, dinv, Wn)
        else:
            feat = _last(h, amean, scale, batch, g, bt, feat)
    return feat, 0


# speedup vs baseline: 9.2826x; 1.0416x over previous
import jax
import jax.numpy as jnp
from jax import lax
from jax.experimental import pallas as pl
from jax.experimental.pallas import tpu as pltpu
from jax.experimental.pallas import tpu_sc as plsc

_N = 10000
_E = 320000
_H = 128
_G = 64
_NC = 2
_NS = 16
_NW = _NC * _NS
_CH = 128
_STEPS = 79
_EPAD = _NW * _STEPS * _CH
_NP = 10240
_RPS = _NP // _NS
_NBUF = 3

_MESH = plsc.VectorSubcoreMesh(core_axis_name="c", subcore_axis_name="s",
                               num_cores=_NC, num_subcores=_NS)
_HIGH = lax.Precision.HIGHEST
_f32 = jnp.float32


_NIR = 4


def _prop_body(u_hbm, src_hbm, dst_hbm, zeros_hbm, out_hbm,
               srcr, dstv, rows, acc, gsem, ssem, isem):
    c = lax.axis_index("c")
    s = lax.axis_index("s")
    w = c * _NS + s
    for t in range(_RPS // _CH):
        pltpu.sync_copy(zeros_hbm, acc.at[pl.ds(s * _RPS + t * _CH, _CH)])
    pltpu.sync_copy(dst_hbm.at[w], dstv)
    plsc.subcore_barrier()
    pltpu.async_copy(src_hbm.at[w, 0], srcr.at[0], isem)
    pltpu.make_async_copy(src_hbm.at[w, 0], srcr.at[0], isem).wait()
    pltpu.async_copy(src_hbm.at[w, 1], srcr.at[1], isem)
    pltpu.async_copy(src_hbm.at[w, 2], srcr.at[2], isem)
    pltpu.async_copy(u_hbm.at[srcr.at[0]], rows.at[0], gsem)

    def step(j, carry):
        p = lax.rem(j, 2)
        pltpu.make_async_copy(u_hbm.at[srcr.at[lax.rem(j, _NIR)]],
                              rows.at[p], gsem).wait()

        @pl.when(j >= 1)
        def _wait_scatter():
            pltpu.make_async_copy(rows.at[1 - p], acc.at[dstv.at[j - 1]],
                                  ssem).wait()

        @pl.when(j + 1 < _STEPS)
        def _next_gather():
            r = lax.rem(j + 1, _NIR)
            pltpu.make_async_copy(src_hbm.at[w, j + 1], srcr.at[r],
                                  isem).wait()
            pltpu.async_copy(u_hbm.at[srcr.at[r]], rows.at[1 - p], gsem)

        @pl.when(j + 3 < _STEPS)
        def _next_idx():
            q = lax.rem(j + 3, _NIR)
            pltpu.async_copy(src_hbm.at[w, j + 3], srcr.at[q], isem)

        pltpu.async_copy(rows.at[p], acc.at[dstv.at[j]], ssem, add=True)
        return carry

    lax.fori_loop(0, _STEPS, step, 0)
    pltpu.make_async_copy(rows.at[(_STEPS - 1) % 2],
                          acc.at[dstv.at[_STEPS - 1]], ssem).wait()
    plsc.subcore_barrier()
    pltpu.sync_copy(acc.at[pl.ds(s * _RPS, _RPS)],
                    out_hbm.at[c, pl.ds(s * _RPS, _RPS)])


_prop = pl.kernel(
    _prop_body,
    out_type=jax.ShapeDtypeStruct((_NC, _NP, _H), _f32),
    mesh=_MESH,
    scratch_types=[
        pltpu.VMEM((_NIR, _CH), jnp.int32),
        pltpu.VMEM((_STEPS, _CH), jnp.int32),
        pltpu.VMEM((2, _CH, _H), _f32),
        pltpu.VMEM_SHARED((_NP, _H), _f32),
        pltpu.SemaphoreType.DMA,
        pltpu.SemaphoreType.DMA,
        pltpu.SemaphoreType.DMA,
    ],
)


def _dot(a, b):
    return lax.dot_general(a, b, (((a.ndim - 1,), (0,)), ((), ())),
                           precision=_HIGH, preferred_element_type=_f32)


def _tdot(a, b):
    return lax.dot_general(a, b, (((0,), (0,)), ((), ())),
                           precision=_HIGH, preferred_element_type=_f32)


def _pre_body(x_ref, we_ref, be_ref, w0_ref, dp_ref, u_ref, dinv_ref):
    deg = dp_ref[0, 0:_N, 0:1] + dp_ref[1, 0:_N, 0:1] + 1.0
    dinv = lax.rsqrt(deg)
    dinv_ref[...] = dinv
    h = _dot(x_ref[...], we_ref[...]) + be_ref[...][None, :]
    u_ref[...] = dinv * _dot(h, w0_ref[...])


def _mid_body(sp_ref, u_ref, dinv_ref, out_ref):
    dinv = dinv_ref[...]
    out_ref[...] = dinv * dinv * (sp_ref[0, 0:_N] + sp_ref[1, 0:_N]
                                  + u_ref[...])


def _onehot(batch_ref):
    return (batch_ref[...][:, None] ==
            lax.broadcasted_iota(jnp.int32, (_N, _G), 1)).astype(_f32)


def _stat_body(sp_ref, u_ref, dinv_ref, batch_ref, bw_ref, a_ref,
               h_ref, amean_ref, scale_ref):
    h = dinv_ref[...] * (sp_ref[0, 0:_N] + sp_ref[1, 0:_N] + u_ref[...]) \
        + bw_ref[...][None, :]
    h_ref[...] = h
    onehot = _onehot(batch_ref)
    cnt = jnp.maximum(jnp.sum(onehot, axis=0), 1.0)[:, None]
    amean = a_ref[...][None, :] * (_tdot(onehot, h) / cnt)
    amean_ref[...] = amean
    sub = h - _dot(onehot, amean)
    var = _tdot(onehot, sub * sub) / cnt
    scale_ref[...] = lax.rsqrt(var + 1e-5)


def _apply_core(h_ref, amean_ref, scale_ref, batch_ref, g_ref, bt_ref,
                fin_ref):
    onehot = _onehot(batch_ref)
    cnt = jnp.maximum(jnp.sum(onehot, axis=0), 1.0)[:, None]
    sub = h_ref[...] - _dot(onehot, amean_ref[...])
    hn = g_ref[...][None, :] * sub * _dot(onehot, scale_ref[...]) \
        + bt_ref[...][None, :]
    hn = jnp.where(hn >= 0, hn, 0.01 * hn)
    feat = _tdot(onehot, hn) / cnt
    return hn, fin_ref[...] + feat * (1.0 / 3.0)


def _apply_body(h_ref, amean_ref, scale_ref, batch_ref, g_ref, bt_ref,
                fin_ref, dinv_ref, wn_ref, fout_ref, unext_ref):
    hn, fout = _apply_core(h_ref, amean_ref, scale_ref, batch_ref, g_ref,
                           bt_ref, fin_ref)
    fout_ref[...] = fout
    unext_ref[...] = dinv_ref[...] * _dot(hn, wn_ref[...])


def _last_body(h_ref, amean_ref, scale_ref, batch_ref, g_ref, bt_ref,
               fin_ref, fout_ref):
    _, fout = _apply_core(h_ref, amean_ref, scale_ref, batch_ref, g_ref,
                          bt_ref, fin_ref)
    fout_ref[...] = fout


_pre = pl.pallas_call(
    _pre_body,
    out_shape=[jax.ShapeDtypeStruct((_N, _H), _f32),
               jax.ShapeDtypeStruct((_N, 1), _f32)])
_mid = pl.pallas_call(
    _mid_body,
    out_shape=jax.ShapeDtypeStruct((_N, _H), _f32))
_stat = pl.pallas_call(
    _stat_body,
    out_shape=[jax.ShapeDtypeStruct((_N, _H), _f32),
               jax.ShapeDtypeStruct((_G, _H), _f32),
               jax.ShapeDtypeStruct((_G, _H), _f32)])
_apply = pl.pallas_call(
    _apply_body,
    out_shape=[jax.ShapeDtypeStruct((_G, _H), _f32),
               jax.ShapeDtypeStruct((_N, _H), _f32)])
_last = pl.pallas_call(
    _last_body,
    out_shape=jax.ShapeDtypeStruct((_G, _H), _f32))


def _prep_edges(edge_index):
    pad = _EPAD - _E
    src = jnp.concatenate([edge_index[0], jnp.zeros((pad,), jnp.int32)])
    dst = jnp.concatenate([edge_index[1],
                           _N + (jnp.arange(pad, dtype=jnp.int32) % (_NP - _N))])
    src3 = src.reshape(_STEPS, _NW, _CH).transpose(1, 0, 2)
    dst3 = dst.reshape(_STEPS, _NW, _CH).transpose(1, 0, 2)
    return src3, dst3


def kernel(x, edge_index, batch, W_emb, b_emb, W0, bW0, g0, bt0, a0,
           W1, bW1, g1, bt1, a1, W2, bW2, g2, bt2, a2):
    src3, dst3 = _prep_edges(edge_index)
    ones_t = jnp.ones((_N, _H), _f32)
    zeros_f = jnp.zeros((_CH, _H), _f32)

    dparts = _prop(ones_t, src3, dst3, zeros_f)
    u, dinv = _pre(x, W_emb, b_emb, W0, dparts)

    feat = jnp.zeros((_G, _H), _f32)
    params = [(bW0, g0, bt0, a0, W1), (bW1, g1, bt1, a1, W2),
              (bW2, g2, bt2, a2, None)]
    for bW, g, bt, a, Wn in params:
        sp = _prop(u, src3, dst3, zeros_f)
        u2 = _mid(sp, u, dinv)
        sp2 = _prop(u2, src3, dst3, zeros_f)
        h, amean, scale = _stat(sp2, u2, dinv, batch, bW, a)
        if Wn is not None:
            feat, u = _apply(h, amean, scale, batch, g, bt, feat, dinv, Wn)
        else:
            feat = _last(h, amean, scale, batch, g, bt, feat)
    return feat, 0


# speedup vs baseline: 12.1104x; 1.3046x over previous
import jax
import jax.numpy as jnp
from jax import lax
from jax.experimental import pallas as pl
from jax.experimental.pallas import tpu as pltpu
from jax.experimental.pallas import tpu_sc as plsc

_N = 10000
_E = 320000
_H = 128
_G = 64
_NC = 2
_NS = 16
_NW = _NC * _NS
_CH = 120
_STEPS = 84
_EPAD = _NW * _STEPS * _CH
_NP = 10240
_RPS = _NP // _NS
_NBUF = 3

_MESH = plsc.VectorSubcoreMesh(core_axis_name="c", subcore_axis_name="s",
                               num_cores=_NC, num_subcores=_NS)
_HIGH = lax.Precision.HIGHEST
_f32 = jnp.float32


_NIR = 4


def _prop_body(u_hbm, src_hbm, dst_hbm, zeros_hbm, out_hbm,
               srcr, dstr, rows, acc, gsem, ssem, isem, dsem):
    c = lax.axis_index("c")
    s = lax.axis_index("s")
    w = c * _NS + s
    pltpu.sync_copy(zeros_hbm, acc.at[pl.ds(s * _RPS, _RPS)])
    for t in range(3):
        pltpu.sync_copy(src_hbm.at[w, t], srcr.at[t])
        pltpu.sync_copy(dst_hbm.at[w, t], dstr.at[t])
    plsc.subcore_barrier()
    for t in range(2):
        pltpu.async_copy(u_hbm.at[srcr.at[t]], rows.at[t], gsem)

    def step(j, carry):
        p = lax.rem(j, 3)
        pltpu.make_async_copy(u_hbm.at[srcr.at[lax.rem(j, _NIR)]],
                              rows.at[p], gsem).wait()

        @pl.when(j >= 3)
        def _wait_dst_idx():
            pltpu.make_async_copy(dst_hbm.at[w, j], dstr.at[lax.rem(j, _NIR)],
                                  dsem).wait()

        pltpu.async_copy(rows.at[p], acc.at[dstr.at[lax.rem(j, _NIR)]],
                         ssem, add=True)

        @pl.when(j >= 1)
        def _wait_prev_scatter():
            q = lax.rem(j - 1, 3)
            pltpu.make_async_copy(rows.at[q],
                                  acc.at[dstr.at[lax.rem(j - 1, _NIR)]],
                                  ssem).wait()

        @pl.when(j + 3 < _STEPS)
        def _prefetch_dst():
            r = lax.rem(j + 3, _NIR)
            pltpu.async_copy(dst_hbm.at[w, j + 3], dstr.at[r], dsem)

        @pl.when(jnp.logical_and(j >= 1, j + 2 < _STEPS))
        def _wait_src_idx():
            pltpu.make_async_copy(src_hbm.at[w, j + 2],
                                  srcr.at[lax.rem(j + 2, _NIR)], isem).wait()

        @pl.when(j + 2 < _STEPS)
        def _next_gather():
            pltpu.async_copy(u_hbm.at[srcr.at[lax.rem(j + 2, _NIR)]],
                             rows.at[lax.rem(j + 2, 3)], gsem)

        @pl.when(j + 3 < _STEPS)
        def _prefetch_src():
            r = lax.rem(j + 3, _NIR)
            pltpu.async_copy(src_hbm.at[w, j + 3], srcr.at[r], isem)

        return carry

    lax.fori_loop(0, _STEPS, step, 0)
    pltpu.make_async_copy(rows.at[(_STEPS - 1) % 3],
                          acc.at[dstr.at[(_STEPS - 1) % _NIR]], ssem).wait()
    plsc.subcore_barrier()
    pltpu.sync_copy(acc.at[pl.ds(s * _RPS, _RPS)],
                    out_hbm.at[c, pl.ds(s * _RPS, _RPS)])


_prop = pl.kernel(
    _prop_body,
    out_type=jax.ShapeDtypeStruct((_NC, _NP, _H), _f32),
    mesh=_MESH,
    scratch_types=[
        pltpu.VMEM((_NIR, _CH), jnp.int32),
        pltpu.VMEM((_NIR, _CH), jnp.int32),
        pltpu.VMEM((3, _CH, _H), _f32),
        pltpu.VMEM_SHARED((_NP, _H), _f32),
        pltpu.SemaphoreType.DMA,
        pltpu.SemaphoreType.DMA,
        pltpu.SemaphoreType.DMA,
        pltpu.SemaphoreType.DMA,
    ],
)


def _dot(a, b):
    return lax.dot_general(a, b, (((a.ndim - 1,), (0,)), ((), ())),
                           precision=_HIGH, preferred_element_type=_f32)


def _tdot(a, b):
    return lax.dot_general(a, b, (((0,), (0,)), ((), ())),
                           precision=_HIGH, preferred_element_type=_f32)


def _pre_body(x_ref, we_ref, be_ref, w0_ref, dp_ref, u_ref, dinv_ref):
    deg = dp_ref[0, 0:_N, 0:1] + dp_ref[1, 0:_N, 0:1] + 1.0
    dinv = lax.rsqrt(deg)
    dinv_ref[...] = dinv
    h = _dot(x_ref[...], we_ref[...]) + be_ref[...][None, :]
    u_ref[...] = dinv * _dot(h, w0_ref[...])


def _mid_body(sp_ref, u_ref, dinv_ref, out_ref):
    dinv = dinv_ref[...]
    out_ref[...] = dinv * dinv * (sp_ref[0, 0:_N] + sp_ref[1, 0:_N]
                                  + u_ref[...])


def _onehot(batch_ref):
    return (batch_ref[...][:, None] ==
            lax.broadcasted_iota(jnp.int32, (_N, _G), 1)).astype(_f32)


def _stat_body(sp_ref, u_ref, dinv_ref, batch_ref, bw_ref, a_ref,
               h_ref, amean_ref, scale_ref):
    h = dinv_ref[...] * (sp_ref[0, 0:_N] + sp_ref[1, 0:_N] + u_ref[...]) \
        + bw_ref[...][None, :]
    h_ref[...] = h
    onehot = _onehot(batch_ref)
    cnt = jnp.maximum(jnp.sum(onehot, axis=0), 1.0)[:, None]
    amean = a_ref[...][None, :] * (_tdot(onehot, h) / cnt)
    amean_ref[...] = amean
    sub = h - _dot(onehot, amean)
    var = _tdot(onehot, sub * sub) / cnt
    scale_ref[...] = lax.rsqrt(var + 1e-5)


def _apply_core(h_ref, amean_ref, scale_ref, batch_ref, g_ref, bt_ref,
                fin_ref):
    onehot = _onehot(batch_ref)
    cnt = jnp.maximum(jnp.sum(onehot, axis=0), 1.0)[:, None]
    sub = h_ref[...] - _dot(onehot, amean_ref[...])
    hn = g_ref[...][None, :] * sub * _dot(onehot, scale_ref[...]) \
        + bt_ref[...][None, :]
    hn = jnp.where(hn >= 0, hn, 0.01 * hn)
    feat = _tdot(onehot, hn) / cnt
    return hn, fin_ref[...] + feat * (1.0 / 3.0)


def _apply_body(h_ref, amean_ref, scale_ref, batch_ref, g_ref, bt_ref,
                fin_ref, dinv_ref, wn_ref, fout_ref, unext_ref):
    hn, fout = _apply_core(h_ref, amean_ref, scale_ref, batch_ref, g_ref,
                           bt_ref, fin_ref)
    fout_ref[...] = fout
    unext_ref[...] = dinv_ref[...] * _dot(hn, wn_ref[...])


def _last_body(h_ref, amean_ref, scale_ref, batch_ref, g_ref, bt_ref,
               fin_ref, fout_ref):
    _, fout = _apply_core(h_ref, amean_ref, scale_ref, batch_ref, g_ref,
                          bt_ref, fin_ref)
    fout_ref[...] = fout


_pre = pl.pallas_call(
    _pre_body,
    out_shape=[jax.ShapeDtypeStruct((_N, _H), _f32),
               jax.ShapeDtypeStruct((_N, 1), _f32)])
_mid = pl.pallas_call(
    _mid_body,
    out_shape=jax.ShapeDtypeStruct((_N, _H), _f32))
_stat = pl.pallas_call(
    _stat_body,
    out_shape=[jax.ShapeDtypeStruct((_N, _H), _f32),
               jax.ShapeDtypeStruct((_G, _H), _f32),
               jax.ShapeDtypeStruct((_G, _H), _f32)])
_apply = pl.pallas_call(
    _apply_body,
    out_shape=[jax.ShapeDtypeStruct((_G, _H), _f32),
               jax.ShapeDtypeStruct((_N, _H), _f32)])
_last = pl.pallas_call(
    _last_body,
    out_shape=jax.ShapeDtypeStruct((_G, _H), _f32))


def _prep_edges(edge_index):
    pad = _EPAD - _E
    src = jnp.concatenate([edge_index[0], jnp.zeros((pad,), jnp.int32)])
    dst = jnp.concatenate([edge_index[1],
                           _N + (jnp.arange(pad, dtype=jnp.int32) % (_NP - _N))])
    src3 = src.reshape(_STEPS, _NW, _CH).transpose(1, 0, 2)
    dst3 = dst.reshape(_STEPS, _NW, _CH).transpose(1, 0, 2)
    return src3, dst3


def kernel(x, edge_index, batch, W_emb, b_emb, W0, bW0, g0, bt0, a0,
           W1, bW1, g1, bt1, a1, W2, bW2, g2, bt2, a2):
    src3, dst3 = _prep_edges(edge_index)
    ones_t = jnp.ones((_N, _H), _f32)
    zeros_f = jnp.zeros((_RPS, _H), _f32)

    dparts = _prop(ones_t, src3, dst3, zeros_f)
    u, dinv = _pre(x, W_emb, b_emb, W0, dparts)

    feat = jnp.zeros((_G, _H), _f32)
    params = [(bW0, g0, bt0, a0, W1), (bW1, g1, bt1, a1, W2),
              (bW2, g2, bt2, a2, None)]
    for bW, g, bt, a, Wn in params:
        sp = _prop(u, src3, dst3, zeros_f)
        u2 = _mid(sp, u, dinv)
        sp2 = _prop(u2, src3, dst3, zeros_f)
        h, amean, scale = _stat(sp2, u2, dinv, batch, bW, a)
        if Wn is not None:
            feat, u = _apply(h, amean, scale, batch, g, bt, feat, dinv, Wn)
        else:
            feat = _last(h, amean, scale, batch, g, bt, feat)
    return feat, 0


# speedup vs baseline: 15.6706x; 1.2940x over previous
import jax
import jax.numpy as jnp
from jax import lax
from jax.experimental import pallas as pl
from jax.experimental.pallas import tpu as pltpu
from jax.experimental.pallas import tpu_sc as plsc

_N = 10000
_E = 320000
_H = 128
_G = 64
_NC = 2
_NS = 16
_NW = _NC * _NS
_CH = 88
_STEPS = 114
_EPAD = _NW * _STEPS * _CH
_NP = 10240
_RPS = _NP // _NS
_NBUF = 3

_MESH = plsc.VectorSubcoreMesh(core_axis_name="c", subcore_axis_name="s",
                               num_cores=_NC, num_subcores=_NS)
_HIGH = lax.Precision.HIGHEST
_f32 = jnp.float32


_NSR = 4
_NDR = 5


def _prop_body(u_hbm, src_hbm, dst_hbm, zeros_hbm, out_hbm,
               srcr, dstr, rows, acc, gsem, ssem, isem, dsem):
    c = lax.axis_index("c")
    s = lax.axis_index("s")
    w = c * _NS + s
    pltpu.sync_copy(zeros_hbm, acc.at[pl.ds(s * _RPS, _RPS)])
    for t in range(3):
        pltpu.sync_copy(src_hbm.at[w, t], srcr.at[t])
        pltpu.sync_copy(dst_hbm.at[w, t], dstr.at[t])
    plsc.subcore_barrier()
    for t in range(2):
        pltpu.async_copy(u_hbm.at[srcr.at[t]], rows.at[t], gsem)

    def step(j, carry):
        p = lax.rem(j, 4)
        pltpu.make_async_copy(u_hbm.at[srcr.at[lax.rem(j, _NSR)]],
                              rows.at[p], gsem).wait()

        @pl.when(j >= 3)
        def _wait_dst_idx():
            pltpu.make_async_copy(dst_hbm.at[w, j], dstr.at[lax.rem(j, _NDR)],
                                  dsem).wait()

        pltpu.async_copy(rows.at[p], acc.at[dstr.at[lax.rem(j, _NDR)]],
                         ssem, add=True)

        @pl.when(j >= 2)
        def _wait_prev_scatter():
            q = lax.rem(j - 2, 4)
            pltpu.make_async_copy(rows.at[q],
                                  acc.at[dstr.at[lax.rem(j - 2, _NDR)]],
                                  ssem).wait()

        @pl.when(j + 3 < _STEPS)
        def _prefetch_dst():
            r = lax.rem(j + 3, _NDR)
            pltpu.async_copy(dst_hbm.at[w, j + 3], dstr.at[r], dsem)

        @pl.when(jnp.logical_and(j >= 1, j + 2 < _STEPS))
        def _wait_src_idx():
            pltpu.make_async_copy(src_hbm.at[w, j + 2],
                                  srcr.at[lax.rem(j + 2, _NSR)], isem).wait()

        @pl.when(j + 2 < _STEPS)
        def _next_gather():
            pltpu.async_copy(u_hbm.at[srcr.at[lax.rem(j + 2, _NSR)]],
                             rows.at[lax.rem(j + 2, 4)], gsem)

        @pl.when(j + 3 < _STEPS)
        def _prefetch_src():
            r = lax.rem(j + 3, _NSR)
            pltpu.async_copy(src_hbm.at[w, j + 3], srcr.at[r], isem)

        return carry

    lax.fori_loop(0, _STEPS, step, 0)
    for j in (_STEPS - 2, _STEPS - 1):
        pltpu.make_async_copy(rows.at[j % 4],
                              acc.at[dstr.at[j % _NDR]], ssem).wait()
    plsc.subcore_barrier()
    pltpu.sync_copy(acc.at[pl.ds(s * _RPS, _RPS)],
                    out_hbm.at[c, pl.ds(s * _RPS, _RPS)])


_prop = pl.kernel(
    _prop_body,
    out_type=jax.ShapeDtypeStruct((_NC, _NP, _H), _f32),
    mesh=_MESH,
    scratch_types=[
        pltpu.VMEM((_NSR, _CH), jnp.int32),
        pltpu.VMEM((_NDR, _CH), jnp.int32),
        pltpu.VMEM((4, _CH, _H), _f32),
        pltpu.VMEM_SHARED((_NP, _H), _f32),
        pltpu.SemaphoreType.DMA,
        pltpu.SemaphoreType.DMA,
        pltpu.SemaphoreType.DMA,
        pltpu.SemaphoreType.DMA,
    ],
)


def _dot(a, b):
    return lax.dot_general(a, b, (((a.ndim - 1,), (0,)), ((), ())),
                           precision=_HIGH, preferred_element_type=_f32)


def _tdot(a, b):
    return lax.dot_general(a, b, (((0,), (0,)), ((), ())),
                           precision=_HIGH, preferred_element_type=_f32)


def _pre_body(x_ref, we_ref, be_ref, w0_ref, dp_ref, u_ref, dinv_ref):
    deg = dp_ref[0, 0:_N, 0:1] + dp_ref[1, 0:_N, 0:1] + 1.0
    dinv = lax.rsqrt(deg)
    dinv_ref[...] = dinv
    h = _dot(x_ref[...], we_ref[...]) + be_ref[...][None, :]
    u_ref[...] = dinv * _dot(h, w0_ref[...])


def _mid_body(sp_ref, u_ref, dinv_ref, out_ref):
    dinv = dinv_ref[...]
    out_ref[...] = dinv * dinv * (sp_ref[0, 0:_N] + sp_ref[1, 0:_N]
                                  + u_ref[...])


def _onehot(batch_ref):
    return (batch_ref[...][:, None] ==
            lax.broadcasted_iota(jnp.int32, (_N, _G), 1)).astype(_f32)


def _stat_body(sp_ref, u_ref, dinv_ref, batch_ref, bw_ref, a_ref,
               h_ref, amean_ref, scale_ref):
    h = dinv_ref[...] * (sp_ref[0, 0:_N] + sp_ref[1, 0:_N] + u_ref[...]) \
        + bw_ref[...][None, :]
    h_ref[...] = h
    onehot = _onehot(batch_ref)
    cnt = jnp.maximum(jnp.sum(onehot, axis=0), 1.0)[:, None]
    amean = a_ref[...][None, :] * (_tdot(onehot, h) / cnt)
    amean_ref[...] = amean
    sub = h - _dot(onehot, amean)
    var = _tdot(onehot, sub * sub) / cnt
    scale_ref[...] = lax.rsqrt(var + 1e-5)


def _apply_core(h_ref, amean_ref, scale_ref, batch_ref, g_ref, bt_ref,
                fin_ref):
    onehot = _onehot(batch_ref)
    cnt = jnp.maximum(jnp.sum(onehot, axis=0), 1.0)[:, None]
    sub = h_ref[...] - _dot(onehot, amean_ref[...])
    hn = g_ref[...][None, :] * sub * _dot(onehot, scale_ref[...]) \
        + bt_ref[...][None, :]
    hn = jnp.where(hn >= 0, hn, 0.01 * hn)
    feat = _tdot(onehot, hn) / cnt
    return hn, fin_ref[...] + feat * (1.0 / 3.0)


def _apply_body(h_ref, amean_ref, scale_ref, batch_ref, g_ref, bt_ref,
                fin_ref, dinv_ref, wn_ref, fout_ref, unext_ref):
    hn, fout = _apply_core(h_ref, amean_ref, scale_ref, batch_ref, g_ref,
                           bt_ref, fin_ref)
    fout_ref[...] = fout
    unext_ref[...] = dinv_ref[...] * _dot(hn, wn_ref[...])


def _last_body(h_ref, amean_ref, scale_ref, batch_ref, g_ref, bt_ref,
               fin_ref, fout_ref):
    _, fout = _apply_core(h_ref, amean_ref, scale_ref, batch_ref, g_ref,
                          bt_ref, fin_ref)
    fout_ref[...] = fout


_pre = pl.pallas_call(
    _pre_body,
    out_shape=[jax.ShapeDtypeStruct((_N, _H), _f32),
               jax.ShapeDtypeStruct((_N, 1), _f32)])
_mid = pl.pallas_call(
    _mid_body,
    out_shape=jax.ShapeDtypeStruct((_N, _H), _f32))
_stat = pl.pallas_call(
    _stat_body,
    out_shape=[jax.ShapeDtypeStruct((_N, _H), _f32),
               jax.ShapeDtypeStruct((_G, _H), _f32),
               jax.ShapeDtypeStruct((_G, _H), _f32)])
_apply = pl.pallas_call(
    _apply_body,
    out_shape=[jax.ShapeDtypeStruct((_G, _H), _f32),
               jax.ShapeDtypeStruct((_N, _H), _f32)])
_last = pl.pallas_call(
    _last_body,
    out_shape=jax.ShapeDtypeStruct((_G, _H), _f32))


def _prep_edges(edge_index):
    pad = _EPAD - _E
    src = jnp.concatenate([edge_index[0], jnp.zeros((pad,), jnp.int32)])
    dst = jnp.concatenate([edge_index[1],
                           _N + (jnp.arange(pad, dtype=jnp.int32) % (_NP - _N))])
    src3 = src.reshape(_STEPS, _NW, _CH).transpose(1, 0, 2)
    dst3 = dst.reshape(_STEPS, _NW, _CH).transpose(1, 0, 2)
    return src3, dst3


def kernel(x, edge_index, batch, W_emb, b_emb, W0, bW0, g0, bt0, a0,
           W1, bW1, g1, bt1, a1, W2, bW2, g2, bt2, a2):
    src3, dst3 = _prep_edges(edge_index)
    ones_t = jnp.ones((_N, _H), _f32)
    zeros_f = jnp.zeros((_RPS, _H), _f32)

    dparts = _prop(ones_t, src3, dst3, zeros_f)
    u, dinv = _pre(x, W_emb, b_emb, W0, dparts)

    feat = jnp.zeros((_G, _H), _f32)
    params = [(bW0, g0, bt0, a0, W1), (bW1, g1, bt1, a1, W2),
              (bW2, g2, bt2, a2, None)]
    for bW, g, bt, a, Wn in params:
        sp = _prop(u, src3, dst3, zeros_f)
        u2 = _mid(sp, u, dinv)
        sp2 = _prop(u2, src3, dst3, zeros_f)
        h, amean, scale = _stat(sp2, u2, dinv, batch, bW, a)
        if Wn is not None:
            feat, u = _apply(h, amean, scale, batch, g, bt, feat, dinv, Wn)
        else:
            feat = _last(h, amean, scale, batch, g, bt, feat)
    return feat, 0


# speedup vs baseline: 18.8268x; 1.2014x over previous
import jax
import jax.numpy as jnp
from jax import lax
from jax.experimental import pallas as pl
from jax.experimental.pallas import tpu as pltpu
from jax.experimental.pallas import tpu_sc as plsc

_N = 10000
_E = 320000
_H = 128
_G = 64
_NC = 2
_NS = 16
_NW = _NC * _NS
_CH = 72
_STEPS = 139
_EPAD = _NW * _STEPS * _CH
_NP = 10240
_RPS = _NP // _NS
_NBUF = 3

_MESH = plsc.VectorSubcoreMesh(core_axis_name="c", subcore_axis_name="s",
                               num_cores=_NC, num_subcores=_NS)
_HIGH = lax.Precision.HIGHEST
_f32 = jnp.float32


_NSR = 5
_NDR = 6


def _prop_body(u_hbm, src_hbm, dst_hbm, zeros_hbm, out_hbm,
               srcr, dstr, rows, acc, gsem, ssem, isem, dsem):
    c = lax.axis_index("c")
    s = lax.axis_index("s")
    w = c * _NS + s
    pltpu.sync_copy(zeros_hbm, acc.at[pl.ds(s * _RPS, _RPS)])
    for t in range(4):
        pltpu.sync_copy(src_hbm.at[w, t], srcr.at[t])
        pltpu.sync_copy(dst_hbm.at[w, t], dstr.at[t])
    plsc.subcore_barrier()
    for t in range(3):
        pltpu.async_copy(u_hbm.at[srcr.at[t]], rows.at[t], gsem)

    def step(j, carry):
        p = lax.rem(j, 5)
        pltpu.make_async_copy(u_hbm.at[srcr.at[lax.rem(j, _NSR)]],
                              rows.at[p], gsem).wait()

        @pl.when(j >= 4)
        def _wait_dst_idx():
            pltpu.make_async_copy(dst_hbm.at[w, j], dstr.at[lax.rem(j, _NDR)],
                                  dsem).wait()

        pltpu.async_copy(rows.at[p], acc.at[dstr.at[lax.rem(j, _NDR)]],
                         ssem, add=True)

        @pl.when(j >= 2)
        def _wait_prev_scatter():
            q = lax.rem(j - 2, 5)
            pltpu.make_async_copy(rows.at[q],
                                  acc.at[dstr.at[lax.rem(j - 2, _NDR)]],
                                  ssem).wait()

        @pl.when(j + 4 < _STEPS)
        def _prefetch_dst():
            r = lax.rem(j + 4, _NDR)
            pltpu.async_copy(dst_hbm.at[w, j + 4], dstr.at[r], dsem)

        @pl.when(jnp.logical_and(j >= 1, j + 3 < _STEPS))
        def _wait_src_idx():
            pltpu.make_async_copy(src_hbm.at[w, j + 3],
                                  srcr.at[lax.rem(j + 3, _NSR)], isem).wait()

        @pl.when(j + 3 < _STEPS)
        def _next_gather():
            pltpu.async_copy(u_hbm.at[srcr.at[lax.rem(j + 3, _NSR)]],
                             rows.at[lax.rem(j + 3, 5)], gsem)

        @pl.when(j + 4 < _STEPS)
        def _prefetch_src():
            r = lax.rem(j + 4, _NSR)
            pltpu.async_copy(src_hbm.at[w, j + 4], srcr.at[r], isem)

        return carry

    lax.fori_loop(0, _STEPS, step, 0)
    for j in (_STEPS - 2, _STEPS - 1):
        pltpu.make_async_copy(rows.at[j % 5],
                              acc.at[dstr.at[j % _NDR]], ssem).wait()
    plsc.subcore_barrier()
    pltpu.sync_copy(acc.at[pl.ds(s * _RPS, _RPS)],
                    out_hbm.at[c, pl.ds(s * _RPS, _RPS)])


_prop = pl.kernel(
    _prop_body,
    out_type=jax.ShapeDtypeStruct((_NC, _NP, _H), _f32),
    mesh=_MESH,
    scratch_types=[
        pltpu.VMEM((_NSR, _CH), jnp.int32),
        pltpu.VMEM((_NDR, _CH), jnp.int32),
        pltpu.VMEM((5, _CH, _H), _f32),
        pltpu.VMEM_SHARED((_NP, _H), _f32),
        pltpu.SemaphoreType.DMA,
        pltpu.SemaphoreType.DMA,
        pltpu.SemaphoreType.DMA,
        pltpu.SemaphoreType.DMA,
    ],
)


def _dot(a, b):
    return lax.dot_general(a, b, (((a.ndim - 1,), (0,)), ((), ())),
                           precision=_HIGH, preferred_element_type=_f32)


def _tdot(a, b):
    return lax.dot_general(a, b, (((0,), (0,)), ((), ())),
                           precision=_HIGH, preferred_element_type=_f32)


def _pre_body(x_ref, we_ref, be_ref, w0_ref, dp_ref, u_ref, dinv_ref):
    deg = dp_ref[0, 0:_N, 0:1] + dp_ref[1, 0:_N, 0:1] + 1.0
    dinv = lax.rsqrt(deg)
    dinv_ref[...] = dinv
    h = _dot(x_ref[...], we_ref[...]) + be_ref[...][None, :]
    u_ref[...] = dinv * _dot(h, w0_ref[...])


def _mid_body(sp_ref, u_ref, dinv_ref, out_ref):
    dinv = dinv_ref[...]
    out_ref[...] = dinv * dinv * (sp_ref[0, 0:_N] + sp_ref[1, 0:_N]
                                  + u_ref[...])


def _onehot(batch_ref):
    return (batch_ref[...][:, None] ==
            lax.broadcasted_iota(jnp.int32, (_N, _G), 1)).astype(_f32)


def _stat_body(sp_ref, u_ref, dinv_ref, batch_ref, bw_ref, a_ref,
               h_ref, amean_ref, scale_ref):
    h = dinv_ref[...] * (sp_ref[0, 0:_N] + sp_ref[1, 0:_N] + u_ref[...]) \
        + bw_ref[...][None, :]
    h_ref[...] = h
    onehot = _onehot(batch_ref)
    cnt = jnp.maximum(jnp.sum(onehot, axis=0), 1.0)[:, None]
    amean = a_ref[...][None, :] * (_tdot(onehot, h) / cnt)
    amean_ref[...] = amean
    sub = h - _dot(onehot, amean)
    var = _tdot(onehot, sub * sub) / cnt
    scale_ref[...] = lax.rsqrt(var + 1e-5)


def _apply_core(h_ref, amean_ref, scale_ref, batch_ref, g_ref, bt_ref,
                fin_ref):
    onehot = _onehot(batch_ref)
    cnt = jnp.maximum(jnp.sum(onehot, axis=0), 1.0)[:, None]
    sub = h_ref[...] - _dot(onehot, amean_ref[...])
    hn = g_ref[...][None, :] * sub * _dot(onehot, scale_ref[...]) \
        + bt_ref[...][None, :]
    hn = jnp.where(hn >= 0, hn, 0.01 * hn)
    feat = _tdot(onehot, hn) / cnt
    return hn, fin_ref[...] + feat * (1.0 / 3.0)


def _apply_body(h_ref, amean_ref, scale_ref, batch_ref, g_ref, bt_ref,
                fin_ref, dinv_ref, wn_ref, fout_ref, unext_ref):
    hn, fout = _apply_core(h_ref, amean_ref, scale_ref, batch_ref, g_ref,
                           bt_ref, fin_ref)
    fout_ref[...] = fout
    unext_ref[...] = dinv_ref[...] * _dot(hn, wn_ref[...])


def _last_body(h_ref, amean_ref, scale_ref, batch_ref, g_ref, bt_ref,
               fin_ref, fout_ref):
    _, fout = _apply_core(h_ref, amean_ref, scale_ref, batch_ref, g_ref,
                          bt_ref, fin_ref)
    fout_ref[...] = fout


_pre = pl.pallas_call(
    _pre_body,
    out_shape=[jax.ShapeDtypeStruct((_N, _H), _f32),
               jax.ShapeDtypeStruct((_N, 1), _f32)])
_mid = pl.pallas_call(
    _mid_body,
    out_shape=jax.ShapeDtypeStruct((_N, _H), _f32))
_stat = pl.pallas_call(
    _stat_body,
    out_shape=[jax.ShapeDtypeStruct((_N, _H), _f32),
               jax.ShapeDtypeStruct((_G, _H), _f32),
               jax.ShapeDtypeStruct((_G, _H), _f32)])
_apply = pl.pallas_call(
    _apply_body,
    out_shape=[jax.ShapeDtypeStruct((_G, _H), _f32),
               jax.ShapeDtypeStruct((_N, _H), _f32)])
_last = pl.pallas_call(
    _last_body,
    out_shape=jax.ShapeDtypeStruct((_G, _H), _f32))


def _prep_edges(edge_index):
    pad = _EPAD - _E
    src = jnp.concatenate([edge_index[0], jnp.zeros((pad,), jnp.int32)])
    dst = jnp.concatenate([edge_index[1],
                           _N + (jnp.arange(pad, dtype=jnp.int32) % (_NP - _N))])
    src3 = src.reshape(_STEPS, _NW, _CH).transpose(1, 0, 2)
    dst3 = dst.reshape(_STEPS, _NW, _CH).transpose(1, 0, 2)
    return src3, dst3


def kernel(x, edge_index, batch, W_emb, b_emb, W0, bW0, g0, bt0, a0,
           W1, bW1, g1, bt1, a1, W2, bW2, g2, bt2, a2):
    src3, dst3 = _prep_edges(edge_index)
    ones_t = jnp.ones((_N, _H), _f32)
    zeros_f = jnp.zeros((_RPS, _H), _f32)

    dparts = _prop(ones_t, src3, dst3, zeros_f)
    u, dinv = _pre(x, W_emb, b_emb, W0, dparts)

    feat = jnp.zeros((_G, _H), _f32)
    params = [(bW0, g0, bt0, a0, W1), (bW1, g1, bt1, a1, W2),
              (bW2, g2, bt2, a2, None)]
    for bW, g, bt, a, Wn in params:
        sp = _prop(u, src3, dst3, zeros_f)
        u2 = _mid(sp, u, dinv)
        sp2 = _prop(u2, src3, dst3, zeros_f)
        h, amean, scale = _stat(sp2, u2, dinv, batch, bW, a)
        if Wn is not None:
            feat, u = _apply(h, amean, scale, batch, g, bt, feat, dinv, Wn)
        else:
            feat = _last(h, amean, scale, batch, g, bt, feat)
    return feat, 0
